# speedup vs baseline: 3.3045x; 3.3045x over previous
import jax
import jax.numpy as jnp
from jax import lax
from jax.experimental import pallas as pl

_K = 64
_N = 32768
_ROWS = 128
_BLOCK_ROWS = 8


def _body(x_ref, o_ref):
    x = x_ref[...]
    ub = lax.bitcast_convert_type(x, jnp.uint32)
    top = jnp.uint32(0x80000000)
    u = jnp.where(ub >= top, ~ub, ub | top)

    def bit_step(i, t):
        b = jnp.uint32(31) - i.astype(jnp.uint32)
        cand = t | lax.shift_left(jnp.uint32(1), b)
        c = jnp.sum((u >= cand).astype(jnp.int32), axis=1, keepdims=True)
        return jnp.where(c >= _K, cand, t)

    t0 = jnp.zeros((_BLOCK_ROWS, 1), jnp.uint32)
    t = lax.fori_loop(0, 32, bit_step, t0)

    gt = u > t
    eq = u == t
    n_gt = jnp.sum(gt.astype(jnp.int32), axis=1, keepdims=True)
    r = _K - n_gt

    idx = lax.broadcasted_iota(jnp.int32, (_BLOCK_ROWS, _N), 1)
    eq_i = eq.astype(jnp.int32)

    def idx_step(i, J):
        b = jnp.int32(14) - i
        cand = J | lax.shift_left(jnp.int32(1), b)
        c = jnp.sum(jnp.where(idx < cand, eq_i, 0), axis=1, keepdims=True)
        return jnp.where(c < r, cand, J)

    J = lax.fori_loop(0, 15, idx_step, jnp.zeros((_BLOCK_ROWS, 1), jnp.int32))

    mask = gt | (eq & (idx <= J))
    o_ref[...] = jnp.where(mask, x, 0.0)


def kernel(x):
    return pl.pallas_call(
        _body,
        grid=(_ROWS // _BLOCK_ROWS,),
        in_specs=[pl.BlockSpec((_BLOCK_ROWS, _N), lambda i: (i, 0))],
        out_specs=pl.BlockSpec((_BLOCK_ROWS, _N), lambda i: (i, 0)),
        out_shape=jax.ShapeDtypeStruct(x.shape, x.dtype),
    )(x)
